# baseline (speedup 1.0000x reference)
import functools

import jax
import jax.numpy as jnp
from jax import lax
from jax.experimental import pallas as pl
from jax.experimental.pallas import tpu as pltpu
from jax.experimental.pallas import tpu_sc as plsc

_B = 32
_C = 192
_H = 56
_W = 56

_info = plsc.get_sparse_core_info()
_NC = _info.num_cores
_NS = _info.num_subcores
_NW = _NC * _NS
_CPW = _C // _NW
_L = _info.num_lanes

_mesh = plsc.VectorSubcoreMesh(core_axis_name="c", subcore_axis_name="s")


@functools.partial(
    pl.kernel,
    mesh=_mesh,
    compiler_params=pltpu.CompilerParams(needs_layout_passes=False),
    out_type=jax.ShapeDtypeStruct((_B, _C, _H, _W), jnp.float32),
    scratch_types=[
        pltpu.VMEM((_C,), jnp.int32),
        pltpu.SemaphoreType.DMA,
    ],
)
def _permute_channels(x_hbm, perm_hbm, out_hbm, perm_v, sem):
    wid = lax.axis_index("s") * _NC + lax.axis_index("c")
    base = wid * _CPW
    pltpu.sync_copy(perm_hbm, perm_v)
    lanes = lax.iota(jnp.int32, _L)
    copies = []
    for j in range(_CPW):
        c = base + j
        vec = perm_v[pl.ds((c // _L) * _L, _L)]
        pc = jnp.sum(jnp.where(lanes == c % _L, vec, 0))
        copies.append(
            pltpu.async_copy(x_hbm.at[:, pc], out_hbm.at[:, c], sem)
        )
    for cp in copies:
        cp.wait()


def kernel(x, perm):
    return _permute_channels(x, perm.astype(jnp.int32))


# speedup vs baseline: 13.1832x; 13.1832x over previous
import functools

import jax
import jax.numpy as jnp
from jax import lax
from jax.experimental import pallas as pl
from jax.experimental.pallas import tpu as pltpu
from jax.experimental.pallas import tpu_sc as plsc

_B = 32
_C = 192
_H = 56
_W = 56

_info = plsc.get_sparse_core_info()
_NC = _info.num_cores
_NS = _info.num_subcores
_NW = _NC * _NS
_CPW = _C // _NW
_L = _info.num_lanes
_BG = 8
_NG = _B // _BG
_NUNIT = _CPW * _NG

_mesh = plsc.VectorSubcoreMesh(core_axis_name="c", subcore_axis_name="s")


@functools.partial(
    pl.kernel,
    mesh=_mesh,
    compiler_params=pltpu.CompilerParams(needs_layout_passes=False),
    out_type=jax.ShapeDtypeStruct((_B, _C, _H, _W), jnp.float32),
    scratch_types=[
        pltpu.VMEM((_C,), jnp.int32),
        pltpu.VMEM((_BG, 1, _H, _W), jnp.float32),
        pltpu.VMEM((_BG, 1, _H, _W), jnp.float32),
        pltpu.SemaphoreType.DMA,
        pltpu.SemaphoreType.DMA,
        pltpu.SemaphoreType.DMA,
        pltpu.SemaphoreType.DMA,
    ],
)
def _permute_channels(x_hbm, perm_hbm, out_hbm, perm_v, buf0, buf1,
                      g0, g1, s0, s1):
    wid = lax.axis_index("s") * _NC + lax.axis_index("c")
    base = wid * _CPW
    pltpu.sync_copy(perm_hbm, perm_v)
    lanes = lax.iota(jnp.int32, _L)

    pcs = []
    for j in range(_CPW):
        c = base + j
        vec = perm_v[pl.ds((c // _L) * _L, _L)]
        pcs.append(jnp.sum(jnp.where(lanes == c % _L, vec, 0)))

    bufs = [buf0, buf1]
    gsems = [g0, g1]
    ssems = [s0, s1]

    def gather_start(u, slot):
        j, g = divmod(u, _NG)
        return pltpu.async_copy(
            x_hbm.at[pl.ds(g * _BG, _BG), pl.ds(pcs[j], 1)],
            bufs[slot], gsems[slot],
        )

    def store_start(u, slot):
        j, g = divmod(u, _NG)
        return pltpu.async_copy(
            bufs[slot],
            out_hbm.at[pl.ds(g * _BG, _BG), pl.ds(base + j, 1)],
            ssems[slot],
        )

    gathers = [gather_start(0, 0), gather_start(1, 1)]
    stores = [None, None]
    for u in range(_NUNIT):
        slot = u & 1
        gathers[slot].wait()
        stores[slot] = store_start(u, slot)
        if u + 2 < _NUNIT:
            stores[slot].wait()
            gathers[slot] = gather_start(u + 2, slot)
    stores[0].wait()
    stores[1].wait()


def kernel(x, perm):
    return _permute_channels(x, perm.astype(jnp.int32))


# speedup vs baseline: 28.6275x; 2.1715x over previous
import functools

import jax
import jax.numpy as jnp
from jax import lax
from jax.experimental import pallas as pl
from jax.experimental.pallas import tpu as pltpu
from jax.experimental.pallas import tpu_sc as plsc

_B = 32
_C = 192
_H = 56
_W = 56
_P = _B * _H * _W

_info = plsc.get_sparse_core_info()
_NC = _info.num_cores
_NS = _info.num_subcores
_NW = _NC * _NS
_L = _info.num_lanes
_NBLK = _C // _L
_PPW = _P // _NW
_CHUNK = 112
_NCHUNK = _PPW // _CHUNK

_mesh = plsc.VectorSubcoreMesh(core_axis_name="c", subcore_axis_name="s")


@functools.partial(
    pl.kernel,
    mesh=_mesh,
    compiler_params=pltpu.CompilerParams(needs_layout_passes=False),
    out_type=jax.ShapeDtypeStruct((_P, _C), jnp.float32),
    scratch_types=[
        pltpu.VMEM((_C,), jnp.int32),
        pltpu.VMEM((_CHUNK, _C), jnp.float32),
        pltpu.VMEM((_CHUNK, _C), jnp.float32),
        pltpu.VMEM((_CHUNK, _C), jnp.float32),
        pltpu.VMEM((_CHUNK, _C), jnp.float32),
        pltpu.SemaphoreType.DMA,
        pltpu.SemaphoreType.DMA,
        pltpu.SemaphoreType.DMA,
        pltpu.SemaphoreType.DMA,
    ],
)
def _permute_lanes(y_hbm, perm_hbm, out_hbm, perm_v, in0, in1, o0, o1,
                   g0, g1, s0, s1):
    wid = lax.axis_index("s") * _NC + lax.axis_index("c")
    base = wid * _PPW
    pltpu.sync_copy(perm_hbm, perm_v)
    pcs = [perm_v[pl.ds(j * _L, _L)] for j in range(_NBLK)]

    ins = [in0, in1]
    outs = [o0, o1]
    gsems = [g0, g1]
    ssems = [s0, s1]

    def gather_start(i, slot):
        return pltpu.async_copy(
            y_hbm.at[pl.ds(base + i * _CHUNK, _CHUNK)], ins[slot],
            gsems[slot],
        )

    def store_start(i, slot):
        return pltpu.async_copy(
            outs[slot], out_hbm.at[pl.ds(base + i * _CHUNK, _CHUNK)],
            ssems[slot],
        )

    def permute_chunk(slot):
        src = ins[slot]
        dst = outs[slot]

        def body(p, _):
            pv = jnp.full((_L,), p, dtype=jnp.int32)
            for j in range(_NBLK):
                dst[p, pl.ds(j * _L, _L)] = plsc.load_gather(
                    src, [pv, pcs[j]]
                )
            return _

        lax.fori_loop(0, _CHUNK, body, None)

    gathers = [gather_start(0, 0), gather_start(1, 1)]
    stores = [None, None]
    for i in range(_NCHUNK):
        slot = i & 1
        gathers[slot].wait()
        if stores[slot] is not None:
            stores[slot].wait()
        permute_chunk(slot)
        stores[slot] = store_start(i, slot)
        if i + 2 < _NCHUNK:
            gathers[slot] = gather_start(i + 2, slot)
    stores[0].wait()
    stores[1].wait()


def kernel(x, perm):
    y = jnp.transpose(x, (0, 2, 3, 1)).reshape(_P, _C)
    out_t = _permute_lanes(y, perm.astype(jnp.int32))
    return jnp.transpose(out_t.reshape(_B, _H, _W, _C), (0, 3, 1, 2))


# speedup vs baseline: 53.6586x; 1.8744x over previous
import functools

import jax
import jax.numpy as jnp
from jax import lax
from jax.experimental import pallas as pl
from jax.experimental.pallas import tpu as pltpu
from jax.experimental.pallas import tpu_sc as plsc

_B = 32
_C = 192
_H = 56
_W = 56
_P = _B * _H * _W

_info = plsc.get_sparse_core_info()
_NC = _info.num_cores
_NS = _info.num_subcores
_NW = _NC * _NS
_L = _info.num_lanes
_NBLK = _C // _L
_PPW = _P // _NW
_CHUNK = 112
_NCHUNK = _PPW // _CHUNK

_mesh = plsc.VectorSubcoreMesh(core_axis_name="c", subcore_axis_name="s")


@functools.partial(
    pl.kernel,
    mesh=_mesh,
    compiler_params=pltpu.CompilerParams(needs_layout_passes=False),
    out_type=jax.ShapeDtypeStruct((_P, _C), jnp.float32),
    scratch_types=[
        pltpu.VMEM((_C,), jnp.int32),
        pltpu.VMEM((_CHUNK, _C), jnp.float32),
        pltpu.VMEM((_CHUNK, _C), jnp.float32),
        pltpu.VMEM((_CHUNK, _C), jnp.float32),
        pltpu.VMEM((_CHUNK, _C), jnp.float32),
        pltpu.SemaphoreType.DMA,
        pltpu.SemaphoreType.DMA,
        pltpu.SemaphoreType.DMA,
        pltpu.SemaphoreType.DMA,
    ],
)
def _permute_lanes(y_hbm, perm_hbm, out_hbm, perm_v, in0, in1, o0, o1,
                   g0, g1, s0, s1):
    wid = lax.axis_index("s") * _NC + lax.axis_index("c")
    base = wid * _PPW
    pltpu.sync_copy(perm_hbm, perm_v)
    pcs = [perm_v[pl.ds(j * _L, _L)] for j in range(_NBLK)]

    ins = [in0, in1]
    outs = [o0, o1]
    gsems = [g0, g1]
    ssems = [s0, s1]

    def gather_start(i, slot):
        return pltpu.async_copy(
            y_hbm.at[pl.ds(base + i * _CHUNK, _CHUNK)], ins[slot],
            gsems[slot],
        )

    def store_start(i, slot):
        return pltpu.async_copy(
            outs[slot], out_hbm.at[pl.ds(base + i * _CHUNK, _CHUNK)],
            ssems[slot],
        )

    def permute_chunk(slot):
        src = ins[slot]
        dst = outs[slot]

        @plsc.parallel_loop(0, _CHUNK, unroll=4)
        def body(p):
            pv = jnp.full((_L,), p, dtype=jnp.int32)
            for j in range(_NBLK):
                dst[p, pl.ds(j * _L, _L)] = plsc.load_gather(
                    src, [pv, pcs[j]]
                )

    gathers = [gather_start(0, 0), gather_start(1, 1)]
    stores = [None, None]
    for i in range(_NCHUNK):
        slot = i & 1
        gathers[slot].wait()
        if stores[slot] is not None:
            stores[slot].wait()
        permute_chunk(slot)
        stores[slot] = store_start(i, slot)
        if i + 2 < _NCHUNK:
            gathers[slot] = gather_start(i + 2, slot)
    stores[0].wait()
    stores[1].wait()


def kernel(x, perm):
    y = jnp.transpose(x, (0, 2, 3, 1)).reshape(_P, _C)
    out_t = _permute_lanes(y, perm.astype(jnp.int32))
    return jnp.transpose(out_t.reshape(_B, _H, _W, _C), (0, 3, 1, 2))
